# baseline (speedup 1.0000x reference)
import functools

import jax
import jax.numpy as jnp
from jax import lax
from jax.experimental import pallas as pl
from jax.experimental.pallas import tpu as pltpu
from jax.experimental.pallas import tpu_sc as plsc

N = 10000
D = 128
E = 320000
NC, NS = 2, 16
NW = NC * NS
EW = E // NW
K = 100
NCH = EW // K
NH = 2
NCH2 = NCH // NH
RD = 624
ZB = 104
RB = 2000


def _proj_body(x_ref, g_ref, b_ref, w1_ref, b1_ref, w2_ref, b2_ref, o_ref):
    x = x_ref[...]
    m = jnp.mean(x, axis=-1, keepdims=True)
    v = jnp.mean((x - m) ** 2, axis=-1, keepdims=True)
    xn = (x - m) * lax.rsqrt(v + 1e-5) * g_ref[...] + b_ref[...]
    h = jnp.maximum(jnp.dot(xn, w1_ref[...], preferred_element_type=jnp.float32)
                    + b1_ref[...], 0.0)
    h = jnp.maximum(jnp.dot(h, w2_ref[...], preferred_element_type=jnp.float32)
                    + b2_ref[...], 0.0)
    o_ref[...] = h


def _proj_tc(x, g, b, w1, b1, w2, b2):
    F = x.shape[1]
    grid = N // RB
    return pl.pallas_call(
        _proj_body,
        grid=(grid,),
        in_specs=[
            pl.BlockSpec((RB, F), lambda i: (i, 0)),
            pl.BlockSpec((1, F), lambda i: (0, 0)),
            pl.BlockSpec((1, F), lambda i: (0, 0)),
            pl.BlockSpec((F, D), lambda i: (0, 0)),
            pl.BlockSpec((1, D), lambda i: (0, 0)),
            pl.BlockSpec((D, D), lambda i: (0, 0)),
            pl.BlockSpec((1, D), lambda i: (0, 0)),
        ],
        out_specs=pl.BlockSpec((RB, D), lambda i: (i, 0)),
        out_shape=jax.ShapeDtypeStruct((N, D), jnp.float32),
    )(x, g.reshape(1, F), b.reshape(1, F), w1, b1.reshape(1, D), w2,
      b2.reshape(1, D))


def _erow_body(lnb_ref, w1_ref, b1_ref, w2_ref, b2_ref, o_ref):
    h = jnp.maximum(lnb_ref[...] * w1_ref[...] + b1_ref[...], 0.0)
    o_ref[...] = jnp.maximum(
        jnp.dot(h, w2_ref[...], preferred_element_type=jnp.float32)
        + b2_ref[...], 0.0)


def _erow_tc(edge_ln_b, w1, b1, w2, b2):
    return pl.pallas_call(
        _erow_body,
        out_shape=jax.ShapeDtypeStruct((1, D), jnp.float32),
    )(edge_ln_b.reshape(1, 1), w1.reshape(1, D), b1.reshape(1, D), w2,
      b2.reshape(1, D))


def _prep_body(r_ref, l_ref, wl_ref, bl_ref, erow_ref, we_ref, wr_ref,
               a_ref, b_ref):
    econst = jnp.dot(erow_ref[...], we_ref[...],
                     preferred_element_type=jnp.float32)
    a_ref[...] = (jnp.dot(r_ref[...], wl_ref[...],
                          preferred_element_type=jnp.float32)
                  + bl_ref[...] + econst)
    b_ref[...] = jnp.dot(l_ref[...], wr_ref[...],
                         preferred_element_type=jnp.float32)


def _prep_tc(right, left, wl, bl, erow, we, wr):
    grid = N // RB
    full = lambda i: (0, 0)
    return pl.pallas_call(
        _prep_body,
        grid=(grid,),
        in_specs=[
            pl.BlockSpec((RB, D), lambda i: (i, 0)),
            pl.BlockSpec((RB, D), lambda i: (i, 0)),
            pl.BlockSpec((D, D), full),
            pl.BlockSpec((1, D), full),
            pl.BlockSpec((1, D), full),
            pl.BlockSpec((D, D), full),
            pl.BlockSpec((D, D), full),
        ],
        out_specs=[
            pl.BlockSpec((RB, D), lambda i: (i, 0)),
            pl.BlockSpec((RB, D), lambda i: (i, 0)),
        ],
        out_shape=[
            jax.ShapeDtypeStruct((N, D), jnp.float32),
            jax.ShapeDtypeStruct((N, D), jnp.float32),
        ],
    )(right, left, wl, bl.reshape(1, D), erow, we, wr)


def _post_body(aggp_ref, degp_ref, r_ref, wf_ref, bf_ref, g_ref, b_ref,
               wo1a_ref, wo1b_ref, bo1_ref, wo2_ref, bo2_ref, o_ref):
    aggm = aggp_ref[0] + aggp_ref[1]
    deg = (degp_ref[0, :, 0:1] + degp_ref[1, :, 0:1])
    agg = (jnp.dot(aggm, wf_ref[...], preferred_element_type=jnp.float32)
           + deg * bf_ref[...])
    m = jnp.mean(agg, axis=-1, keepdims=True)
    v = jnp.mean((agg - m) ** 2, axis=-1, keepdims=True)
    agg = (agg - m) * lax.rsqrt(v + 1e-5) * g_ref[...] + b_ref[...]
    h = jnp.maximum(
        jnp.dot(agg, wo1a_ref[...], preferred_element_type=jnp.float32)
        + jnp.dot(r_ref[...], wo1b_ref[...], preferred_element_type=jnp.float32)
        + bo1_ref[...], 0.0)
    o_ref[...] = (jnp.dot(h, wo2_ref[...], preferred_element_type=jnp.float32)
                  + bo2_ref[...])


def _post_tc(aggp, degp, right, wf, bf, lnp_g, lnp_b, wo1, bo1, wo2, bo2):
    grid = N // RB
    full = lambda i: (0, 0)
    return pl.pallas_call(
        _post_body,
        grid=(grid,),
        in_specs=[
            pl.BlockSpec((NC, RB, D), lambda i: (0, i, 0)),
            pl.BlockSpec((NC, RB, 16), lambda i: (0, i, 0)),
            pl.BlockSpec((RB, D), lambda i: (i, 0)),
            pl.BlockSpec((D, D), full),
            pl.BlockSpec((1, D), full),
            pl.BlockSpec((1, D), full),
            pl.BlockSpec((1, D), full),
            pl.BlockSpec((D, D), full),
            pl.BlockSpec((D, D), full),
            pl.BlockSpec((1, D), full),
            pl.BlockSpec((D, D), full),
            pl.BlockSpec((1, D), full),
        ],
        out_specs=pl.BlockSpec((RB, D), lambda i: (i, 0)),
        out_shape=jax.ShapeDtypeStruct((N, D), jnp.float32),
    )(aggp, degp, right, wf, bf.reshape(1, D), lnp_g.reshape(1, D),
      lnp_b.reshape(1, D), wo1[:D], wo1[D:], bo1.reshape(1, D), wo2,
      bo2.reshape(1, D))


_MESH = plsc.VectorSubcoreMesh(core_axis_name="c", subcore_axis_name="s")


_GDN = lax.GatherDimensionNumbers(offset_dims=(), collapsed_slice_dims=(0,),
                                  start_index_map=(0,))


def _lane_sum16(x):
    for sh in (1, 2, 4, 8):
        perm = lax.iota(jnp.int32, 16) ^ sh
        x = x + lax.gather(x, perm[:, None], _GDN, (1,),
                           mode=lax.GatherScatterMode.PROMISE_IN_BOUNDS)
    return x


def _rsqrt16(x):
    bits = lax.bitcast_convert_type(x, jnp.int32)
    y = lax.bitcast_convert_type(jnp.int32(0x5F3759DF) - (bits >> 1),
                                 jnp.float32)
    for _ in range(3):
        y = y * (1.5 - 0.5 * x * y * y)
    return y


def _fill_zero(buf, rows, cols):
    z = jnp.zeros((16,), jnp.float32)

    def body(i, _):
        for k in range(cols // 16):
            buf[i, pl.ds(k * 16, 16)] = z
        return 0

    lax.fori_loop(0, rows, body, 0)


@functools.partial(
    pl.kernel,
    mesh=_MESH,
    out_type=jax.ShapeDtypeStruct((NC, N, D), jnp.float32),
    scratch_types=[
        pltpu.VMEM((NCH2, 1, K), jnp.int32),
        pltpu.VMEM((NCH2, 1, K), jnp.int32),
        pltpu.VMEM((K, D), jnp.float32),
        pltpu.VMEM((K, D), jnp.float32),
        pltpu.VMEM((D,), jnp.float32),
        pltpu.VMEM((D,), jnp.float32),
        pltpu.VMEM_SHARED((N, D), jnp.float32),
        pltpu.SemaphoreType.DMA,
        pltpu.SemaphoreType.DMA,
    ],
)
def _sc_conv(a_hbm, b_hbm, dst_hbm, src_hbm, g_hbm, bb_hbm, out_hbm,
             dst_v, src_v, buf_a, buf_b, g_v, b_v, acc, sem_a, sem_b):
    c = lax.axis_index("c")
    s = lax.axis_index("s")

    pltpu.sync_copy(g_hbm, g_v)
    pltpu.sync_copy(bb_hbm, b_v)
    gvs = [g_v[pl.ds(k * 16, 16)] for k in range(D // 16)]
    bvs = [b_v[pl.ds(k * 16, 16)] for k in range(D // 16)]

    _fill_zero(buf_a, K, D)
    for q in range(RD // ZB):
        pltpu.sync_copy(buf_a.at[pl.ds(0, ZB)],
                        acc.at[pl.ds(s * RD + q * ZB, ZB)])

    @pl.when(s == NS - 1)
    def _():
        pltpu.sync_copy(buf_a.at[pl.ds(0, N - NS * RD)],
                        acc.at[pl.ds(NS * RD, N - NS * RD)])

    plsc.subcore_barrier()

    inv_d = 1.0 / D

    def chunk(j, _):
        ha = pltpu.async_copy(a_hbm.at[dst_v.at[j, 0]], buf_a, sem_a)
        hb = pltpu.async_copy(b_hbm.at[src_v.at[j, 0]], buf_b, sem_b)
        ha.wait()
        hb.wait()

        @plsc.parallel_loop(0, K, unroll=4)
        def row(i):
            t = [buf_a[i, pl.ds(k * 16, 16)] + buf_b[i, pl.ds(k * 16, 16)]
                 for k in range(D // 16)]
            tot = t[0]
            for k in range(1, D // 16):
                tot = tot + t[k]
            mean = _lane_sum16(tot) * inv_d
            dvs = [t[k] - mean for k in range(D // 16)]
            sq = [dvs[k] * dvs[k] for k in range(D // 16)]
            ssq = sq[0]
            for k in range(1, D // 16):
                ssq = ssq + sq[k]
            rs = _rsqrt16(_lane_sum16(ssq) * inv_d + 1e-5)
            for k in range(D // 16):
                y = dvs[k] * (rs * gvs[k]) + bvs[k]
                buf_a[i, pl.ds(k * 16, 16)] = jnp.maximum(y, 0.0)

        pltpu.sync_copy(buf_a, acc.at[dst_v.at[j, 0]], add=True)
        return 0

    for h in range(NH):
        pltpu.sync_copy(dst_hbm.at[c, s, h], dst_v)
        pltpu.sync_copy(src_hbm.at[c, s, h], src_v)
        lax.fori_loop(0, NCH2, chunk, 0)
    plsc.subcore_barrier()
    pltpu.sync_copy(acc.at[pl.ds(s * RD, RD)],
                    out_hbm.at[c, pl.ds(s * RD, RD)])

    @pl.when(s == NS - 1)
    def _():
        pltpu.sync_copy(acc.at[pl.ds(NS * RD, N - NS * RD)],
                        out_hbm.at[c, pl.ds(NS * RD, N - NS * RD)])


@functools.partial(
    pl.kernel,
    mesh=_MESH,
    out_type=jax.ShapeDtypeStruct((NC, 2, N, 16), jnp.float32),
    scratch_types=[
        pltpu.VMEM((NCH, K), jnp.int32),
        pltpu.VMEM((NCH, K), jnp.int32),
        pltpu.VMEM((K, 16), jnp.float32),
        pltpu.VMEM_SHARED((N, 16), jnp.float32),
        pltpu.VMEM_SHARED((N, 16), jnp.float32),
    ],
)
def _sc_deg(ci_hbm, vi_hbm, out_hbm, ci_v, vi_v, ones_v, acc_c, acc_v):
    c = lax.axis_index("c")
    s = lax.axis_index("s")

    pltpu.sync_copy(ci_hbm.at[c, s], ci_v)
    pltpu.sync_copy(vi_hbm.at[c, s], vi_v)

    _fill_zero(ones_v, K, 16)
    for q in range(RD // ZB):
        pltpu.sync_copy(ones_v.at[pl.ds(0, ZB)],
                        acc_c.at[pl.ds(s * RD + q * ZB, ZB)])
        pltpu.sync_copy(ones_v.at[pl.ds(0, ZB)],
                        acc_v.at[pl.ds(s * RD + q * ZB, ZB)])

    @pl.when(s == NS - 1)
    def _():
        pltpu.sync_copy(ones_v.at[pl.ds(0, N - NS * RD)],
                        acc_c.at[pl.ds(NS * RD, N - NS * RD)])
        pltpu.sync_copy(ones_v.at[pl.ds(0, N - NS * RD)],
                        acc_v.at[pl.ds(NS * RD, N - NS * RD)])

    one = jnp.ones((16,), jnp.float32)

    def fill1(i, _):
        ones_v[i, pl.ds(0, 16)] = one
        return 0

    lax.fori_loop(0, K, fill1, 0)
    plsc.subcore_barrier()

    def chunk(j, _):
        pltpu.sync_copy(ones_v, acc_c.at[ci_v.at[j]], add=True)
        pltpu.sync_copy(ones_v, acc_v.at[vi_v.at[j]], add=True)
        return 0

    lax.fori_loop(0, NCH, chunk, 0)
    plsc.subcore_barrier()
    pltpu.sync_copy(acc_c.at[pl.ds(s * RD, RD)],
                    out_hbm.at[c, 0, pl.ds(s * RD, RD)])
    pltpu.sync_copy(acc_v.at[pl.ds(s * RD, RD)],
                    out_hbm.at[c, 1, pl.ds(s * RD, RD)])

    @pl.when(s == NS - 1)
    def _():
        pltpu.sync_copy(acc_c.at[pl.ds(NS * RD, N - NS * RD)],
                        out_hbm.at[c, 0, pl.ds(NS * RD, N - NS * RD)])
        pltpu.sync_copy(acc_v.at[pl.ds(NS * RD, N - NS * RD)],
                        out_hbm.at[c, 1, pl.ds(NS * RD, N - NS * RD)])


def _conv(left, right, dst4, src4, degp, erow, wl, bl, we, wr, lnf_g, lnf_b,
          wf, bf, lnp_g, lnp_b, wo1, bo1, wo2, bo2):
    a, b = _prep_tc(right, left, wl, bl, erow, we, wr)
    aggp = _sc_conv(a, b, dst4, src4, lnf_g, lnf_b)
    return _post_tc(aggp, degp, right, wf, bf, lnp_g, lnp_b, wo1, bo1, wo2,
                    bo2)


def kernel(constraint_features, variable_features, edge_features, edge_indices, cons_ln_g, cons_ln_b, cons_w1, cons_b1, cons_w2, cons_b2, var_ln_g, var_ln_b, var_w1, var_b1, var_w2, var_b2, edge_ln_g, edge_ln_b, edge_w1, edge_b1, edge_w2, edge_b2, vc1_wl, vc1_bl, vc1_we, vc1_wr, vc1_lnf_g, vc1_lnf_b, vc1_wf, vc1_bf, vc1_lnp_g, vc1_lnp_b, vc1_wo1, vc1_bo1, vc1_wo2, vc1_bo2, cv1_wl, cv1_bl, cv1_we, cv1_wr, cv1_lnf_g, cv1_lnf_b, cv1_wf, cv1_bf, cv1_lnp_g, cv1_lnp_b, cv1_wo1, cv1_bo1, cv1_wo2, cv1_bo2, vc2_wl, vc2_bl, vc2_we, vc2_wr, vc2_lnf_g, vc2_lnf_b, vc2_wf, vc2_bf, vc2_lnp_g, vc2_lnp_b, vc2_wo1, vc2_bo1, vc2_wo2, vc2_bo2, cv2_wl, cv2_bl, cv2_we, cv2_wr, cv2_lnf_g, cv2_lnf_b, cv2_wf, cv2_bf, cv2_lnp_g, cv2_lnp_b, cv2_wo1, cv2_bo1, cv2_wo2, cv2_bo2):
    ci5 = edge_indices[0].reshape(NC, NS, NH, NCH2, 1, K)
    vi5 = edge_indices[1].reshape(NC, NS, NH, NCH2, 1, K)
    ci4 = edge_indices[0].reshape(NC, NS, NCH, K)
    vi4 = edge_indices[1].reshape(NC, NS, NCH, K)

    c0 = _proj_tc(constraint_features, cons_ln_g, cons_ln_b, cons_w1,
                  cons_b1, cons_w2, cons_b2)
    v0 = _proj_tc(variable_features, var_ln_g, var_ln_b, var_w1, var_b1,
                  var_w2, var_b2)
    erow = _erow_tc(edge_ln_b, edge_w1, edge_b1, edge_w2, edge_b2)

    degp = _sc_deg(ci4, vi4)
    degp_c = degp[:, 0]
    degp_v = degp[:, 1]

    c1 = _conv(v0, c0, ci5, vi5, degp_c, erow, vc1_wl, vc1_bl, vc1_we,
               vc1_wr, vc1_lnf_g, vc1_lnf_b, vc1_wf, vc1_bf, vc1_lnp_g,
               vc1_lnp_b, vc1_wo1, vc1_bo1, vc1_wo2, vc1_bo2)
    v1 = _conv(c1, v0, vi5, ci5, degp_v, erow, cv1_wl, cv1_bl, cv1_we,
               cv1_wr, cv1_lnf_g, cv1_lnf_b, cv1_wf, cv1_bf, cv1_lnp_g,
               cv1_lnp_b, cv1_wo1, cv1_bo1, cv1_wo2, cv1_bo2)
    c2 = _conv(v1, c1, ci5, vi5, degp_c, erow, vc2_wl, vc2_bl, vc2_we,
               vc2_wr, vc2_lnf_g, vc2_lnf_b, vc2_wf, vc2_bf, vc2_lnp_g,
               vc2_lnp_b, vc2_wo1, vc2_bo1, vc2_wo2, vc2_bo2)
    v2 = _conv(c2, v1, vi5, ci5, degp_v, erow, cv2_wl, cv2_bl, cv2_we,
               cv2_wr, cv2_lnf_g, cv2_lnf_b, cv2_wf, cv2_bf, cv2_lnp_g,
               cv2_lnp_b, cv2_wo1, cv2_bo1, cv2_wo2, cv2_bo2)

    return jnp.concatenate([c2, v2], axis=0)


# speedup vs baseline: 1.2164x; 1.2164x over previous
import functools

import jax
import jax.numpy as jnp
from jax import lax
from jax.experimental import pallas as pl
from jax.experimental.pallas import tpu as pltpu
from jax.experimental.pallas import tpu_sc as plsc

N = 10000
D = 128
E = 320000
NC, NS = 2, 16
NW = NC * NS
EW = E // NW
K = 100
NCH = EW // K
NH = 2
NCH2 = NCH // NH
RD = 624
ZB = 104
RB = 2000


def _proj_body(x_ref, g_ref, b_ref, w1_ref, b1_ref, w2_ref, b2_ref, o_ref):
    x = x_ref[...]
    m = jnp.mean(x, axis=-1, keepdims=True)
    v = jnp.mean((x - m) ** 2, axis=-1, keepdims=True)
    xn = (x - m) * lax.rsqrt(v + 1e-5) * g_ref[...] + b_ref[...]
    h = jnp.maximum(jnp.dot(xn, w1_ref[...], preferred_element_type=jnp.float32)
                    + b1_ref[...], 0.0)
    h = jnp.maximum(jnp.dot(h, w2_ref[...], preferred_element_type=jnp.float32)
                    + b2_ref[...], 0.0)
    o_ref[...] = h


def _proj_tc(x, g, b, w1, b1, w2, b2):
    F = x.shape[1]
    grid = N // RB
    return pl.pallas_call(
        _proj_body,
        grid=(grid,),
        in_specs=[
            pl.BlockSpec((RB, F), lambda i: (i, 0)),
            pl.BlockSpec((1, F), lambda i: (0, 0)),
            pl.BlockSpec((1, F), lambda i: (0, 0)),
            pl.BlockSpec((F, D), lambda i: (0, 0)),
            pl.BlockSpec((1, D), lambda i: (0, 0)),
            pl.BlockSpec((D, D), lambda i: (0, 0)),
            pl.BlockSpec((1, D), lambda i: (0, 0)),
        ],
        out_specs=pl.BlockSpec((RB, D), lambda i: (i, 0)),
        out_shape=jax.ShapeDtypeStruct((N, D), jnp.float32),
    )(x, g.reshape(1, F), b.reshape(1, F), w1, b1.reshape(1, D), w2,
      b2.reshape(1, D))


def _erow_body(lnb_ref, w1_ref, b1_ref, w2_ref, b2_ref, o_ref):
    h = jnp.maximum(lnb_ref[...] * w1_ref[...] + b1_ref[...], 0.0)
    o_ref[...] = jnp.maximum(
        jnp.dot(h, w2_ref[...], preferred_element_type=jnp.float32)
        + b2_ref[...], 0.0)


def _erow_tc(edge_ln_b, w1, b1, w2, b2):
    return pl.pallas_call(
        _erow_body,
        out_shape=jax.ShapeDtypeStruct((1, D), jnp.float32),
    )(edge_ln_b.reshape(1, 1), w1.reshape(1, D), b1.reshape(1, D), w2,
      b2.reshape(1, D))


def _prep_body(r_ref, l_ref, wl_ref, bl_ref, erow_ref, we_ref, wr_ref,
               a_ref, b_ref):
    econst = jnp.dot(erow_ref[...], we_ref[...],
                     preferred_element_type=jnp.float32)
    a_ref[...] = (jnp.dot(r_ref[...], wl_ref[...],
                          preferred_element_type=jnp.float32)
                  + bl_ref[...] + econst)
    b_ref[...] = jnp.dot(l_ref[...], wr_ref[...],
                         preferred_element_type=jnp.float32)


def _prep_tc(right, left, wl, bl, erow, we, wr):
    grid = N // RB
    full = lambda i: (0, 0)
    return pl.pallas_call(
        _prep_body,
        grid=(grid,),
        in_specs=[
            pl.BlockSpec((RB, D), lambda i: (i, 0)),
            pl.BlockSpec((RB, D), lambda i: (i, 0)),
            pl.BlockSpec((D, D), full),
            pl.BlockSpec((1, D), full),
            pl.BlockSpec((1, D), full),
            pl.BlockSpec((D, D), full),
            pl.BlockSpec((D, D), full),
        ],
        out_specs=[
            pl.BlockSpec((RB, D), lambda i: (i, 0)),
            pl.BlockSpec((RB, D), lambda i: (i, 0)),
        ],
        out_shape=[
            jax.ShapeDtypeStruct((N, D), jnp.float32),
            jax.ShapeDtypeStruct((N, D), jnp.float32),
        ],
    )(right, left, wl, bl.reshape(1, D), erow, we, wr)


def _post_body(aggp_ref, degp_ref, r_ref, wf_ref, bf_ref, g_ref, b_ref,
               wo1a_ref, wo1b_ref, bo1_ref, wo2_ref, bo2_ref, o_ref):
    aggm = aggp_ref[0] + aggp_ref[1]
    deg = (degp_ref[0, :, 0:1] + degp_ref[1, :, 0:1])
    agg = (jnp.dot(aggm, wf_ref[...], preferred_element_type=jnp.float32)
           + deg * bf_ref[...])
    m = jnp.mean(agg, axis=-1, keepdims=True)
    v = jnp.mean((agg - m) ** 2, axis=-1, keepdims=True)
    agg = (agg - m) * lax.rsqrt(v + 1e-5) * g_ref[...] + b_ref[...]
    h = jnp.maximum(
        jnp.dot(agg, wo1a_ref[...], preferred_element_type=jnp.float32)
        + jnp.dot(r_ref[...], wo1b_ref[...], preferred_element_type=jnp.float32)
        + bo1_ref[...], 0.0)
    o_ref[...] = (jnp.dot(h, wo2_ref[...], preferred_element_type=jnp.float32)
                  + bo2_ref[...])


def _post_tc(aggp, degp, right, wf, bf, lnp_g, lnp_b, wo1, bo1, wo2, bo2):
    grid = N // RB
    full = lambda i: (0, 0)
    return pl.pallas_call(
        _post_body,
        grid=(grid,),
        in_specs=[
            pl.BlockSpec((NC, RB, D), lambda i: (0, i, 0)),
            pl.BlockSpec((NC, RB, 16), lambda i: (0, i, 0)),
            pl.BlockSpec((RB, D), lambda i: (i, 0)),
            pl.BlockSpec((D, D), full),
            pl.BlockSpec((1, D), full),
            pl.BlockSpec((1, D), full),
            pl.BlockSpec((1, D), full),
            pl.BlockSpec((D, D), full),
            pl.BlockSpec((D, D), full),
            pl.BlockSpec((1, D), full),
            pl.BlockSpec((D, D), full),
            pl.BlockSpec((1, D), full),
        ],
        out_specs=pl.BlockSpec((RB, D), lambda i: (i, 0)),
        out_shape=jax.ShapeDtypeStruct((N, D), jnp.float32),
    )(aggp, degp, right, wf, bf.reshape(1, D), lnp_g.reshape(1, D),
      lnp_b.reshape(1, D), wo1[:D], wo1[D:], bo1.reshape(1, D), wo2,
      bo2.reshape(1, D))


_MESH = plsc.VectorSubcoreMesh(core_axis_name="c", subcore_axis_name="s")


_GDN = lax.GatherDimensionNumbers(offset_dims=(), collapsed_slice_dims=(0,),
                                  start_index_map=(0,))


def _lane_sum16(x):
    for sh in (1, 2, 4, 8):
        perm = lax.iota(jnp.int32, 16) ^ sh
        x = x + lax.gather(x, perm[:, None], _GDN, (1,),
                           mode=lax.GatherScatterMode.PROMISE_IN_BOUNDS)
    return x


def _rsqrt16(x):
    bits = lax.bitcast_convert_type(x, jnp.int32)
    y = lax.bitcast_convert_type(jnp.int32(0x5F3759DF) - (bits >> 1),
                                 jnp.float32)
    for _ in range(3):
        y = y * (1.5 - 0.5 * x * y * y)
    return y


def _fill_zero(buf, rows, cols):
    z = jnp.zeros((16,), jnp.float32)

    def body(i, _):
        for k in range(cols // 16):
            buf[i, pl.ds(k * 16, 16)] = z
        return 0

    lax.fori_loop(0, rows, body, 0)


@functools.partial(
    pl.kernel,
    mesh=_MESH,
    out_type=jax.ShapeDtypeStruct((NC, N, D), jnp.float32),
    scratch_types=[
        pltpu.VMEM((NCH2, 1, K), jnp.int32),
        pltpu.VMEM((NCH2, 1, K), jnp.int32),
        pltpu.VMEM((K, D), jnp.float32),
        pltpu.VMEM((K, D), jnp.float32),
        pltpu.VMEM((D,), jnp.float32),
        pltpu.VMEM((D,), jnp.float32),
        pltpu.VMEM_SHARED((N, D), jnp.float32),
        pltpu.SemaphoreType.DMA,
        pltpu.SemaphoreType.DMA,
    ],
)
def _sc_conv(a_hbm, b_hbm, dst_hbm, src_hbm, g_hbm, bb_hbm, out_hbm,
             dst_v, src_v, buf_a, buf_b, g_v, b_v, acc, sem_a, sem_b):
    c = lax.axis_index("c")
    s = lax.axis_index("s")

    pltpu.sync_copy(g_hbm, g_v)
    pltpu.sync_copy(bb_hbm, b_v)
    gvs = [g_v[pl.ds(k * 16, 16)] for k in range(D // 16)]
    bvs = [b_v[pl.ds(k * 16, 16)] for k in range(D // 16)]

    _fill_zero(buf_a, K, D)
    for q in range(RD // ZB):
        pltpu.sync_copy(buf_a.at[pl.ds(0, ZB)],
                        acc.at[pl.ds(s * RD + q * ZB, ZB)])

    @pl.when(s == NS - 1)
    def _():
        pltpu.sync_copy(buf_a.at[pl.ds(0, N - NS * RD)],
                        acc.at[pl.ds(NS * RD, N - NS * RD)])

    plsc.subcore_barrier()

    inv_d = 1.0 / D

    def chunk(j, _):
        ha = pltpu.async_copy(a_hbm.at[dst_v.at[j, 0]], buf_a, sem_a)
        hb = pltpu.async_copy(b_hbm.at[src_v.at[j, 0]], buf_b, sem_b)
        ha.wait()
        hb.wait()

        @plsc.parallel_loop(0, K, unroll=2)
        def row(i):
            t = [buf_a[i, pl.ds(k * 16, 16)] + buf_b[i, pl.ds(k * 16, 16)]
                 for k in range(D // 16)]
            tot = t[0]
            for k in range(1, D // 16):
                tot = tot + t[k]
            mean = _lane_sum16(tot) * inv_d
            dvs = [t[k] - mean for k in range(D // 16)]
            sq = [dvs[k] * dvs[k] for k in range(D // 16)]
            ssq = sq[0]
            for k in range(1, D // 16):
                ssq = ssq + sq[k]
            rs = _rsqrt16(_lane_sum16(ssq) * inv_d + 1e-5)
            for k in range(D // 16):
                y = dvs[k] * (rs * gvs[k]) + bvs[k]
                buf_a[i, pl.ds(k * 16, 16)] = jnp.maximum(y, 0.0)

        pltpu.sync_copy(buf_a, acc.at[dst_v.at[j, 0]], add=True)
        return 0

    for h in range(NH):
        pltpu.sync_copy(dst_hbm.at[c, s, h], dst_v)
        pltpu.sync_copy(src_hbm.at[c, s, h], src_v)
        lax.fori_loop(0, NCH2, chunk, 0)
    plsc.subcore_barrier()
    pltpu.sync_copy(acc.at[pl.ds(s * RD, RD)],
                    out_hbm.at[c, pl.ds(s * RD, RD)])

    @pl.when(s == NS - 1)
    def _():
        pltpu.sync_copy(acc.at[pl.ds(NS * RD, N - NS * RD)],
                        out_hbm.at[c, pl.ds(NS * RD, N - NS * RD)])


@functools.partial(
    pl.kernel,
    mesh=_MESH,
    out_type=jax.ShapeDtypeStruct((NC, 2, N, 16), jnp.float32),
    scratch_types=[
        pltpu.VMEM((NCH, K), jnp.int32),
        pltpu.VMEM((NCH, K), jnp.int32),
        pltpu.VMEM((K, 16), jnp.float32),
        pltpu.VMEM_SHARED((N, 16), jnp.float32),
        pltpu.VMEM_SHARED((N, 16), jnp.float32),
    ],
)
def _sc_deg(ci_hbm, vi_hbm, out_hbm, ci_v, vi_v, ones_v, acc_c, acc_v):
    c = lax.axis_index("c")
    s = lax.axis_index("s")

    pltpu.sync_copy(ci_hbm.at[c, s], ci_v)
    pltpu.sync_copy(vi_hbm.at[c, s], vi_v)

    _fill_zero(ones_v, K, 16)
    for q in range(RD // ZB):
        pltpu.sync_copy(ones_v.at[pl.ds(0, ZB)],
                        acc_c.at[pl.ds(s * RD + q * ZB, ZB)])
        pltpu.sync_copy(ones_v.at[pl.ds(0, ZB)],
                        acc_v.at[pl.ds(s * RD + q * ZB, ZB)])

    @pl.when(s == NS - 1)
    def _():
        pltpu.sync_copy(ones_v.at[pl.ds(0, N - NS * RD)],
                        acc_c.at[pl.ds(NS * RD, N - NS * RD)])
        pltpu.sync_copy(ones_v.at[pl.ds(0, N - NS * RD)],
                        acc_v.at[pl.ds(NS * RD, N - NS * RD)])

    one = jnp.ones((16,), jnp.float32)

    def fill1(i, _):
        ones_v[i, pl.ds(0, 16)] = one
        return 0

    lax.fori_loop(0, K, fill1, 0)
    plsc.subcore_barrier()

    def chunk(j, _):
        pltpu.sync_copy(ones_v, acc_c.at[ci_v.at[j]], add=True)
        pltpu.sync_copy(ones_v, acc_v.at[vi_v.at[j]], add=True)
        return 0

    lax.fori_loop(0, NCH, chunk, 0)
    plsc.subcore_barrier()
    pltpu.sync_copy(acc_c.at[pl.ds(s * RD, RD)],
                    out_hbm.at[c, 0, pl.ds(s * RD, RD)])
    pltpu.sync_copy(acc_v.at[pl.ds(s * RD, RD)],
                    out_hbm.at[c, 1, pl.ds(s * RD, RD)])

    @pl.when(s == NS - 1)
    def _():
        pltpu.sync_copy(acc_c.at[pl.ds(NS * RD, N - NS * RD)],
                        out_hbm.at[c, 0, pl.ds(NS * RD, N - NS * RD)])
        pltpu.sync_copy(acc_v.at[pl.ds(NS * RD, N - NS * RD)],
                        out_hbm.at[c, 1, pl.ds(NS * RD, N - NS * RD)])


def _conv(left, right, dst4, src4, degp, erow, wl, bl, we, wr, lnf_g, lnf_b,
          wf, bf, lnp_g, lnp_b, wo1, bo1, wo2, bo2):
    a, b = _prep_tc(right, left, wl, bl, erow, we, wr)
    aggp = _sc_conv(a, b, dst4, src4, lnf_g, lnf_b)
    return _post_tc(aggp, degp, right, wf, bf, lnp_g, lnp_b, wo1, bo1, wo2,
                    bo2)


def kernel(constraint_features, variable_features, edge_features, edge_indices, cons_ln_g, cons_ln_b, cons_w1, cons_b1, cons_w2, cons_b2, var_ln_g, var_ln_b, var_w1, var_b1, var_w2, var_b2, edge_ln_g, edge_ln_b, edge_w1, edge_b1, edge_w2, edge_b2, vc1_wl, vc1_bl, vc1_we, vc1_wr, vc1_lnf_g, vc1_lnf_b, vc1_wf, vc1_bf, vc1_lnp_g, vc1_lnp_b, vc1_wo1, vc1_bo1, vc1_wo2, vc1_bo2, cv1_wl, cv1_bl, cv1_we, cv1_wr, cv1_lnf_g, cv1_lnf_b, cv1_wf, cv1_bf, cv1_lnp_g, cv1_lnp_b, cv1_wo1, cv1_bo1, cv1_wo2, cv1_bo2, vc2_wl, vc2_bl, vc2_we, vc2_wr, vc2_lnf_g, vc2_lnf_b, vc2_wf, vc2_bf, vc2_lnp_g, vc2_lnp_b, vc2_wo1, vc2_bo1, vc2_wo2, vc2_bo2, cv2_wl, cv2_bl, cv2_we, cv2_wr, cv2_lnf_g, cv2_lnf_b, cv2_wf, cv2_bf, cv2_lnp_g, cv2_lnp_b, cv2_wo1, cv2_bo1, cv2_wo2, cv2_bo2):
    ci5 = edge_indices[0].reshape(NC, NS, NH, NCH2, 1, K)
    vi5 = edge_indices[1].reshape(NC, NS, NH, NCH2, 1, K)
    ci4 = edge_indices[0].reshape(NC, NS, NCH, K)
    vi4 = edge_indices[1].reshape(NC, NS, NCH, K)

    c0 = _proj_tc(constraint_features, cons_ln_g, cons_ln_b, cons_w1,
                  cons_b1, cons_w2, cons_b2)
    v0 = _proj_tc(variable_features, var_ln_g, var_ln_b, var_w1, var_b1,
                  var_w2, var_b2)
    erow = _erow_tc(edge_ln_b, edge_w1, edge_b1, edge_w2, edge_b2)

    degp = _sc_deg(ci4, vi4)
    degp_c = degp[:, 0]
    degp_v = degp[:, 1]

    c1 = _conv(v0, c0, ci5, vi5, degp_c, erow, vc1_wl, vc1_bl, vc1_we,
               vc1_wr, vc1_lnf_g, vc1_lnf_b, vc1_wf, vc1_bf, vc1_lnp_g,
               vc1_lnp_b, vc1_wo1, vc1_bo1, vc1_wo2, vc1_bo2)
    v1 = _conv(c1, v0, vi5, ci5, degp_v, erow, cv1_wl, cv1_bl, cv1_we,
               cv1_wr, cv1_lnf_g, cv1_lnf_b, cv1_wf, cv1_bf, cv1_lnp_g,
               cv1_lnp_b, cv1_wo1, cv1_bo1, cv1_wo2, cv1_bo2)
    c2 = _conv(v1, c1, ci5, vi5, degp_c, erow, vc2_wl, vc2_bl, vc2_we,
               vc2_wr, vc2_lnf_g, vc2_lnf_b, vc2_wf, vc2_bf, vc2_lnp_g,
               vc2_lnp_b, vc2_wo1, vc2_bo1, vc2_wo2, vc2_bo2)
    v2 = _conv(c2, v1, vi5, ci5, degp_v, erow, cv2_wl, cv2_bl, cv2_we,
               cv2_wr, cv2_lnf_g, cv2_lnf_b, cv2_wf, cv2_bf, cv2_lnp_g,
               cv2_lnp_b, cv2_wo1, cv2_bo1, cv2_wo2, cv2_bo2)

    return jnp.concatenate([c2, v2], axis=0)
